# baseline (speedup 1.0000x reference)
import jax
import jax.numpy as jnp
from jax.experimental import pallas as pl

N_USERS = 10000
N_HOPS = 2


def _l2_normalize(x):
    n = jnp.linalg.norm(x, axis=-1, keepdims=True)
    return x / jnp.maximum(n, 1e-12)


def _scatter_softmax(scores, index, num_segments):
    seg_max = jax.ops.segment_max(scores, index, num_segments=num_segments)
    seg_max = jnp.where(jnp.isfinite(seg_max), seg_max, 0.0)
    ex = jnp.exp(scores - seg_max[index])
    seg_sum = jax.ops.segment_sum(ex, index, num_segments=num_segments)
    return ex / seg_sum[index]


def _touch(x):
    def body(x_ref, o_ref):
        o_ref[...] = x_ref[...]
    return pl.pallas_call(body, out_shape=jax.ShapeDtypeStruct(x.shape, x.dtype))(x)


def kernel(user_emb, item_emb, edge_index, edge_type, inter_edge, inter_edge_w, relation_emb):
    head = edge_index[0]
    tail = edge_index[1]
    entity_emb = item_emb
    for _ in range(N_HOPS):
        head_emb = entity_emb[head]
        tail_emb = entity_emb[tail]
        rel_emb = relation_emb[edge_type - 1]
        scores = jnp.exp(jnp.sum(head_emb * rel_emb * tail_emb, axis=1))
        attn = _scatter_softmax(scores, head, entity_emb.shape[0])
        entity_emb = jax.ops.segment_sum(attn[:, None] * tail_emb, head,
                                         num_segments=entity_emb.shape[0])
        entity_emb = _l2_normalize(entity_emb)
    item_agg = inter_edge_w[:, None] * entity_emb[inter_edge[1, :]]
    user_agg = jax.ops.segment_sum(item_agg, inter_edge[0, :], num_segments=N_USERS)
    user_out = _l2_normalize(user_agg)
    return (_touch(user_out), entity_emb)


# speedup vs baseline: 1.3688x; 1.3688x over previous
import functools

import jax
import jax.numpy as jnp
from jax import lax
from jax.experimental import pallas as pl
from jax.experimental.pallas import tpu as pltpu
from jax.experimental.pallas import tpu_sc as plsc

NC, NS, L = 2, 16, 16
NW = NC * NS
N_ENT = 10000
N_USR = 10000
C = 128
NR = 12
NP = 10240
STRIPE = NP // NS
CH = 128

_mesh = plsc.VectorSubcoreMesh(core_axis_name="c", subcore_axis_name="s")


def _pad_to(x, n, fill):
    return jnp.concatenate(
        [x, jnp.full((n - x.shape[0],), fill, x.dtype)]) if x.shape[0] != n else x


def _fill_loop(ref, n, value):
    def body(i, _):
        ref[pl.ds(i * L, L)] = jnp.full((L,), value, jnp.float32)
        return 0
    lax.fori_loop(0, n // L, body, 0)


def _scale_rows(rows, w_v):
    iota = lax.iota(jnp.int32, L)

    def grp(g, _):
        sl = pl.ds(g * L, L)
        w16 = w_v[sl]
        e16 = iota + g * L

        def c_body(c, _):
            c16 = jnp.full((L,), 0, jnp.int32) + c
            v = plsc.load_gather(rows, [e16, c16])
            plsc.store_scatter(rows, [e16, c16], v * w16)
            return 0
        lax.fori_loop(0, C, c_body, 0)
        return 0
    lax.fori_loop(0, CH // L, grp, 0)


def _seg_run_max(k16, v16):
    ks, vs = plsc.sort_key_val(k16, v16)
    iota = lax.iota(jnp.int32, L)
    for d in (1, 2, 4, 8):
        idx = jnp.maximum(iota - d, 0)
        kp = ks.at[idx].get(mode="promise_in_bounds")
        vp = vs.at[idx].get(mode="promise_in_bounds")
        same = (kp == ks) & (iota >= d)
        vs = jnp.where(same, jnp.maximum(vs, vp), vs)
    nxt = jnp.minimum(iota + 1, L - 1)
    kn = ks.at[nxt].get(mode="promise_in_bounds")
    is_last = (kn != ks) | (iota == L - 1)
    return ks, vs, is_last


def _make_ka(ep):
    ept = ep // NW
    n_chunks = ept // CH

    @functools.partial(
        pl.kernel,
        out_type=(jax.ShapeDtypeStruct((ep,), jnp.float32),
                  jax.ShapeDtypeStruct((NC, NP), jnp.float32)),
        mesh=_mesh,
        compiler_params=pltpu.CompilerParams(needs_layout_passes=False),
        scratch_types=[
            pltpu.VMEM((CH, C), jnp.float32),
            pltpu.VMEM((CH, C), jnp.float32),
            pltpu.VMEM((NR - 1, C), jnp.float32),
            pltpu.VMEM((CH,), jnp.int32),
            pltpu.VMEM((CH,), jnp.int32),
            pltpu.VMEM((CH,), jnp.int32),
            pltpu.VMEM((CH,), jnp.int32),
            pltpu.VMEM((CH,), jnp.float32),
            pltpu.VMEM((NP,), jnp.float32),
            pltpu.VMEM_SHARED((NS, NP), jnp.float32),
            pltpu.VMEM((NS, STRIPE), jnp.float32),
            pltpu.VMEM((STRIPE,), jnp.float32),
        ],
    )
    def ka(ent, hgat, hseg, tgat, ridx, rel, scores_o, segmax_o,
           hrow, trow, relv, hgat_v, hseg_v, tgat_v, ridx_v, score_v,
           maxarr, smax_sh, redbuf, resbuf):
        cid = lax.axis_index("c")
        sid = lax.axis_index("s")
        wid = cid * NS + sid
        base = wid * ept
        pltpu.sync_copy(rel, relv)
        _fill_loop(maxarr, NP, -jnp.inf)

        def chunk_body(ci, _):
            off = base + ci * CH
            pltpu.sync_copy(hgat.at[pl.ds(off, CH)], hgat_v)
            pltpu.sync_copy(hseg.at[pl.ds(off, CH)], hseg_v)
            pltpu.sync_copy(tgat.at[pl.ds(off, CH)], tgat_v)
            pltpu.sync_copy(ridx.at[pl.ds(off, CH)], ridx_v)
            pltpu.sync_copy(ent.at[hgat_v], hrow)
            pltpu.sync_copy(ent.at[tgat_v], trow)

            iota = lax.iota(jnp.int32, L)

            def grp_scores(g, _):
                sl = pl.ds(g * L, L)
                e16 = iota + g * L
                r16 = ridx_v[sl]

                def c_body(c, acc):
                    c16 = jnp.full((L,), 0, jnp.int32) + c
                    h = plsc.load_gather(hrow, [e16, c16])
                    t = plsc.load_gather(trow, [e16, c16])
                    rv = plsc.load_gather(relv, [r16, c16])
                    return acc + h * t * rv
                dot = lax.fori_loop(0, C, c_body,
                                    jnp.zeros((L,), jnp.float32))
                score_v[sl] = jnp.exp(dot)
                return 0
            lax.fori_loop(0, CH // L, grp_scores, 0)
            pltpu.sync_copy(score_v, scores_o.at[pl.ds(off, CH)])

            def g_body(g, _):
                sl = pl.ds(g * L, L)
                ks, vs, is_last = _seg_run_max(hseg_v[sl], score_v[sl])
                cur = plsc.load_gather(maxarr, [ks], mask=is_last)
                plsc.store_scatter(maxarr, [ks], jnp.maximum(vs, cur),
                                   mask=is_last)
                return 0
            lax.fori_loop(0, CH // L, g_body, 0)
            return 0
        lax.fori_loop(0, n_chunks, chunk_body, 0)

        pltpu.sync_copy(maxarr, smax_sh.at[sid])
        plsc.subcore_barrier()
        pltpu.sync_copy(smax_sh.at[:, pl.ds(sid * STRIPE, STRIPE)], redbuf)

        def red_body(i, _):
            sl = pl.ds(i * L, L)
            m = redbuf[0, sl]
            for t in range(1, NS):
                m = jnp.maximum(m, redbuf[t, sl])
            resbuf[sl] = m
            return 0
        lax.fori_loop(0, STRIPE // L, red_body, 0)
        pltpu.sync_copy(resbuf, segmax_o.at[cid, pl.ds(sid * STRIPE, STRIPE)])

    return ka


def _make_kb(ep):
    ept = ep // NW
    n_chunks = ept // CH

    @functools.partial(
        pl.kernel,
        out_type=(jax.ShapeDtypeStruct((ep,), jnp.float32),
                  jax.ShapeDtypeStruct((NC, NP), jnp.float32)),
        mesh=_mesh,
        compiler_params=pltpu.CompilerParams(needs_layout_passes=False),
        scratch_types=[
            pltpu.VMEM((NP,), jnp.float32),
            pltpu.VMEM((NP,), jnp.float32),
            pltpu.VMEM((CH,), jnp.float32),
            pltpu.VMEM((CH,), jnp.int32),
            pltpu.VMEM((CH,), jnp.float32),
            pltpu.VMEM((STRIPE,), jnp.float32),
            pltpu.VMEM_SHARED((NP,), jnp.float32),
        ],
    )
    def kb(scores, hseg, segmax, ex_o, segsum_o,
           gmax, mx2, score_v, hseg_v, ex_v, stripe_v, ssum_sh):
        cid = lax.axis_index("c")
        sid = lax.axis_index("s")
        wid = cid * NS + sid
        base = wid * ept
        pltpu.sync_copy(segmax.at[0], gmax)
        pltpu.sync_copy(segmax.at[1], mx2)

        def mb(i, _):
            sl = pl.ds(i * L, L)
            gmax[sl] = jnp.maximum(gmax[sl], mx2[sl])
            return 0
        lax.fori_loop(0, NP // L, mb, 0)

        _fill_loop(stripe_v, STRIPE, 0.0)
        pltpu.sync_copy(stripe_v, ssum_sh.at[pl.ds(sid * STRIPE, STRIPE)])
        plsc.subcore_barrier()

        def chunk_body(ci, _):
            off = base + ci * CH
            pltpu.sync_copy(scores.at[pl.ds(off, CH)], score_v)
            pltpu.sync_copy(hseg.at[pl.ds(off, CH)], hseg_v)

            def g_body(g, _):
                sl = pl.ds(g * L, L)
                mx = plsc.load_gather(gmax, [hseg_v[sl]])
                ex_v[sl] = jnp.exp(score_v[sl] - mx)
                return 0
            lax.fori_loop(0, CH // L, g_body, 0)
            pltpu.sync_copy(ex_v, ex_o.at[pl.ds(off, CH)])
            pltpu.sync_copy(ex_v, ssum_sh.at[hseg_v], add=True)
            return 0
        lax.fori_loop(0, n_chunks, chunk_body, 0)

        plsc.subcore_barrier()
        pltpu.sync_copy(ssum_sh.at[pl.ds(sid * STRIPE, STRIPE)],
                        segsum_o.at[cid, pl.ds(sid * STRIPE, STRIPE)])

    return kb


def _make_kc(ep, n_out):
    ept = ep // NW
    n_chunks = ept // CH
    tail_rows = n_out - (NS - 1) * STRIPE

    @functools.partial(
        pl.kernel,
        out_type=jax.ShapeDtypeStruct((NC, n_out, C), jnp.float32),
        mesh=_mesh,
        compiler_params=pltpu.CompilerParams(needs_layout_passes=False),
        scratch_types=[
            pltpu.VMEM((CH, C), jnp.float32),
            pltpu.VMEM((CH,), jnp.int32),
            pltpu.VMEM((CH,), jnp.int32),
            pltpu.VMEM((CH,), jnp.float32),
            pltpu.VMEM_SHARED((NP, C), jnp.float32),
        ],
    )
    def kc(w_in, seg_in, gat_in, ent, part_o,
           rows, seg_v, gat_v, w_v, accum):
        cid = lax.axis_index("c")
        sid = lax.axis_index("s")
        wid = cid * NS + sid
        base = wid * ept

        def zrow(e, _):
            for j in range(C // L):
                rows[e, pl.ds(j * L, L)] = jnp.zeros((L,), jnp.float32)
            return 0
        lax.fori_loop(0, CH, zrow, 0)
        for k in range(STRIPE // CH):
            pltpu.sync_copy(rows, accum.at[pl.ds(sid * STRIPE + k * CH, CH), :])
        plsc.subcore_barrier()

        def chunk_body(ci, _):
            off = base + ci * CH
            pltpu.sync_copy(seg_in.at[pl.ds(off, CH)], seg_v)
            pltpu.sync_copy(gat_in.at[pl.ds(off, CH)], gat_v)
            pltpu.sync_copy(w_in.at[pl.ds(off, CH)], w_v)
            pltpu.sync_copy(ent.at[gat_v], rows)
            _scale_rows(rows, w_v)
            pltpu.sync_copy(rows, accum.at[seg_v], add=True)
            return 0
        lax.fori_loop(0, n_chunks, chunk_body, 0)

        plsc.subcore_barrier()

        @pl.when(sid < NS - 1)
        def _():
            pltpu.sync_copy(accum.at[pl.ds(sid * STRIPE, STRIPE), :],
                            part_o.at[cid, pl.ds(sid * STRIPE, STRIPE), :])

        @pl.when(sid == NS - 1)
        def _():
            pltpu.sync_copy(
                accum.at[pl.ds((NS - 1) * STRIPE, tail_rows), :],
                part_o.at[cid, pl.ds((NS - 1) * STRIPE, tail_rows), :])

    return kc


def _make_kc_attn(ep):
    ept = ep // NW
    n_chunks = ept // CH
    n_out = N_ENT
    tail_rows = n_out - (NS - 1) * STRIPE

    @functools.partial(
        pl.kernel,
        out_type=jax.ShapeDtypeStruct((NC, n_out, C), jnp.float32),
        mesh=_mesh,
        compiler_params=pltpu.CompilerParams(needs_layout_passes=False),
        scratch_types=[
            pltpu.VMEM((NP,), jnp.float32),
            pltpu.VMEM((NP,), jnp.float32),
            pltpu.VMEM((CH, C), jnp.float32),
            pltpu.VMEM((CH,), jnp.int32),
            pltpu.VMEM((CH,), jnp.int32),
            pltpu.VMEM((CH,), jnp.float32),
            pltpu.VMEM((CH,), jnp.float32),
            pltpu.VMEM_SHARED((NP, C), jnp.float32),
        ],
    )
    def kc(ex_in, seg_in, gat_in, segsum, ent, part_o,
           ssum, s2, rows, seg_v, gat_v, ex_v, w_v, accum):
        cid = lax.axis_index("c")
        sid = lax.axis_index("s")
        wid = cid * NS + sid
        base = wid * ept
        pltpu.sync_copy(segsum.at[0], ssum)
        pltpu.sync_copy(segsum.at[1], s2)

        def mb(i, _):
            sl = pl.ds(i * L, L)
            ssum[sl] = ssum[sl] + s2[sl]
            return 0
        lax.fori_loop(0, NP // L, mb, 0)

        def zrow(e, _):
            for j in range(C // L):
                rows[e, pl.ds(j * L, L)] = jnp.zeros((L,), jnp.float32)
            return 0
        lax.fori_loop(0, CH, zrow, 0)
        for k in range(STRIPE // CH):
            pltpu.sync_copy(rows, accum.at[pl.ds(sid * STRIPE + k * CH, CH), :])
        plsc.subcore_barrier()

        def chunk_body(ci, _):
            off = base + ci * CH
            pltpu.sync_copy(seg_in.at[pl.ds(off, CH)], seg_v)
            pltpu.sync_copy(gat_in.at[pl.ds(off, CH)], gat_v)
            pltpu.sync_copy(ex_in.at[pl.ds(off, CH)], ex_v)
            pltpu.sync_copy(ent.at[gat_v], rows)

            def g_body(g, _):
                sl = pl.ds(g * L, L)
                ss = plsc.load_gather(ssum, [seg_v[sl]])
                w_v[sl] = ex_v[sl] / ss
                return 0
            lax.fori_loop(0, CH // L, g_body, 0)
            _scale_rows(rows, w_v)
            pltpu.sync_copy(rows, accum.at[seg_v], add=True)
            return 0
        lax.fori_loop(0, n_chunks, chunk_body, 0)

        plsc.subcore_barrier()

        @pl.when(sid < NS - 1)
        def _():
            pltpu.sync_copy(accum.at[pl.ds(sid * STRIPE, STRIPE), :],
                            part_o.at[cid, pl.ds(sid * STRIPE, STRIPE), :])

        @pl.when(sid == NS - 1)
        def _():
            pltpu.sync_copy(
                accum.at[pl.ds((NS - 1) * STRIPE, tail_rows), :],
                part_o.at[cid, pl.ds((NS - 1) * STRIPE, tail_rows), :])

    return kc


def _norm_combine(parts):
    n = parts.shape[1]
    br = 400
    assert n % br == 0

    def body(p_ref, o_ref):
        s = p_ref[0] + p_ref[1]
        ss = jnp.sum(s * s, axis=1, keepdims=True)
        nrm = jnp.sqrt(ss)
        o_ref[...] = s / jnp.maximum(nrm, 1e-12)

    return pl.pallas_call(
        body,
        grid=(n // br,),
        in_specs=[pl.BlockSpec((NC, br, C), lambda i: (0, i, 0))],
        out_specs=pl.BlockSpec((br, C), lambda i: (i, 0)),
        out_shape=jax.ShapeDtypeStruct((n, C), jnp.float32),
    )(parts)


def kernel(user_emb, item_emb, edge_index, edge_type, inter_edge,
           inter_edge_w, relation_emb):
    e = edge_index.shape[1]
    ep = ((e + NW * CH - 1) // (NW * CH)) * NW * CH
    ni = inter_edge.shape[1]
    nip = ((ni + NW * CH - 1) // (NW * CH)) * NW * CH

    head = edge_index[0]
    tail = edge_index[1]
    hseg = _pad_to(head, ep, N_ENT)
    hgat = _pad_to(head, ep, 0)
    tgat = _pad_to(tail, ep, 0)
    ridx = _pad_to((edge_type + NR - 2) % (NR - 1), ep, 0)

    dseg = _pad_to(inter_edge[0], nip, N_USR)
    sgat = _pad_to(inter_edge[1], nip, 0)
    w_i = _pad_to(inter_edge_w, nip, 0.0)

    ka = _make_ka(ep)
    kb = _make_kb(ep)
    kc = _make_kc_attn(ep)
    ke = _make_kc(nip, N_USR)

    entity = item_emb
    for _ in range(2):
        scores, segmax = ka(entity, hgat, hseg, tgat, ridx, relation_emb)
        ex, segsum = kb(scores, hseg, segmax)
        parts = kc(ex, hseg, tgat, segsum, entity)
        entity = _norm_combine(parts)

    uparts = ke(w_i, dseg, sgat, entity)
    user_out = _norm_combine(uparts)
    return (user_out, entity)


# speedup vs baseline: 1.5808x; 1.1549x over previous
import functools

import jax
import jax.numpy as jnp
from jax import lax
from jax.experimental import pallas as pl
from jax.experimental.pallas import tpu as pltpu
from jax.experimental.pallas import tpu_sc as plsc

NC, NS, L = 2, 16, 16
NW = NC * NS
N_ENT = 10000
N_USR = 10000
C = 128
NR = 12
NP = 10240
STRIPE = NP // NS
CH = 128
BATCH = 16

_mesh = plsc.VectorSubcoreMesh(core_axis_name="c", subcore_axis_name="s")
_params = pltpu.CompilerParams(needs_layout_passes=False)


def _pad_to(x, n, fill):
    return jnp.concatenate(
        [x, jnp.full((n - x.shape[0],), fill, x.dtype)]) if x.shape[0] != n else x


def _fill_loop(ref, n, value):
    def body(i, _):
        ref[pl.ds(i * L, L)] = jnp.full((L,), value, jnp.float32)
        return 0
    lax.fori_loop(0, n // L, body, 0)


def _scale_rows(rows, w_v):
    iota = lax.iota(jnp.int32, L)

    def grp(g, _):
        sl = pl.ds(g * L, L)
        w16 = w_v[sl]
        e16 = iota + g * L

        def c_body(c, _):
            c16 = jnp.full((L,), 0, jnp.int32) + c
            v = plsc.load_gather(rows, [e16, c16])
            plsc.store_scatter(rows, [e16, c16], v * w16)
            return 0
        lax.fori_loop(0, C, c_body, 0)
        return 0
    lax.fori_loop(0, CH // L, grp, 0)


def _seg_run_max(k16, v16):
    ks, vs = plsc.sort_key_val(k16, v16)
    iota = lax.iota(jnp.int32, L)
    for d in (1, 2, 4, 8):
        idx = jnp.maximum(iota - d, 0)
        kp = ks.at[idx].get(mode="promise_in_bounds")
        vp = vs.at[idx].get(mode="promise_in_bounds")
        same = (kp == ks) & (iota >= d)
        vs = jnp.where(same, jnp.maximum(vs, vp), vs)
    nxt = jnp.minimum(iota + 1, L - 1)
    kn = ks.at[nxt].get(mode="promise_in_bounds")
    is_last = (kn != ks) | (iota == L - 1)
    return ks, vs, is_last


def _make_ka(ep):
    ept = ep // NW
    ncht = ept // CH
    npairs = (ncht + 1) // 2

    @functools.partial(
        pl.kernel,
        out_type=(jax.ShapeDtypeStruct((ep,), jnp.float32),
                  jax.ShapeDtypeStruct((NC, NP), jnp.float32)),
        mesh=_mesh,
        compiler_params=_params,
        scratch_types=[
            pltpu.VMEM((ept,), jnp.int32),
            pltpu.VMEM((ept,), jnp.int32),
            pltpu.VMEM((ept,), jnp.int32),
            pltpu.VMEM((ept,), jnp.int32),
            pltpu.VMEM((NR - 1, C), jnp.float32),
            pltpu.VMEM((CH, C), jnp.float32),
            pltpu.VMEM((CH, C), jnp.float32),
            pltpu.VMEM((CH, C), jnp.float32),
            pltpu.VMEM((CH, C), jnp.float32),
            pltpu.VMEM((CH,), jnp.float32),
            pltpu.VMEM((NP,), jnp.float32),
            pltpu.VMEM_SHARED((NS, NP), jnp.float32),
            pltpu.VMEM((STRIPE,), jnp.float32),
            pltpu.VMEM((STRIPE,), jnp.float32),
            pltpu.SemaphoreType.DMA,
            pltpu.SemaphoreType.DMA,
            pltpu.SemaphoreType.DMA,
            pltpu.SemaphoreType.DMA,
        ],
    )
    def ka(ent, hgat, hseg, tgat, ridx, rel, scores_o, segmax_o,
           hgat_sl, hseg_sl, tgat_sl, ridx_sl, relv, hb0, hb1, tb0, tb1,
           score_v, maxarr, smax_sh, tmpbuf, resbuf, hs0, hs1, ts0, ts1):
        cid = lax.axis_index("c")
        sid = lax.axis_index("s")
        wid = cid * NS + sid
        base = wid * ept
        hbufs, tbufs = (hb0, hb1), (tb0, tb1)
        hsems, tsems = (hs0, hs1), (ts0, ts1)

        pltpu.sync_copy(rel, relv)
        pltpu.sync_copy(hgat.at[pl.ds(base, ept)], hgat_sl)
        pltpu.sync_copy(hseg.at[pl.ds(base, ept)], hseg_sl)
        pltpu.sync_copy(tgat.at[pl.ds(base, ept)], tgat_sl)
        pltpu.sync_copy(ridx.at[pl.ds(base, ept)], ridx_sl)
        _fill_loop(maxarr, NP, -jnp.inf)

        def issue(ci, b):
            pltpu.async_copy(ent.at[hgat_sl.at[pl.ds(ci * CH, CH)]],
                             hbufs[b], hsems[b])
            pltpu.async_copy(ent.at[tgat_sl.at[pl.ds(ci * CH, CH)]],
                             tbufs[b], tsems[b])

        issue(0, 0)
        issue(1, 1)
        iota = lax.iota(jnp.int32, L)

        def pair_body(p, _):
            for b in range(2):
                ci = p * 2 + b

                @pl.when(ci < ncht)
                def _():
                    pltpu.make_async_copy(
                        ent.at[hgat_sl.at[pl.ds(ci * CH, CH)]],
                        hbufs[b], hsems[b]).wait()
                    pltpu.make_async_copy(
                        ent.at[tgat_sl.at[pl.ds(ci * CH, CH)]],
                        tbufs[b], tsems[b]).wait()

                    def grp_scores(g, _):
                        sl = pl.ds(g * L, L)
                        e16 = iota + g * L
                        r16 = ridx_sl[pl.ds(ci * CH + g * L, L)]

                        def c_body(c, acc):
                            c16 = jnp.full((L,), 0, jnp.int32) + c
                            h = plsc.load_gather(hbufs[b], [e16, c16])
                            t = plsc.load_gather(tbufs[b], [e16, c16])
                            rv = plsc.load_gather(relv, [r16, c16])
                            return acc + h * t * rv
                        dot = lax.fori_loop(0, C, c_body,
                                            jnp.zeros((L,), jnp.float32))
                        score_v[sl] = jnp.exp(dot)
                        return 0
                    lax.fori_loop(0, CH // L, grp_scores, 0)
                    pltpu.sync_copy(score_v,
                                    scores_o.at[pl.ds(base + ci * CH, CH)])

                    def g_body(g, _):
                        sl = pl.ds(g * L, L)
                        k16 = hseg_sl[pl.ds(ci * CH + g * L, L)]
                        ks, vs, is_last = _seg_run_max(k16, score_v[sl])
                        cur = plsc.load_gather(maxarr, [ks], mask=is_last)
                        plsc.store_scatter(maxarr, [ks],
                                           jnp.maximum(vs, cur), mask=is_last)
                        return 0
                    lax.fori_loop(0, CH // L, g_body, 0)

                    @pl.when(ci + 2 < ncht)
                    def _():
                        issue(ci + 2, b)
            return 0
        lax.fori_loop(0, npairs, pair_body, 0)

        pltpu.sync_copy(maxarr, smax_sh.at[sid])
        plsc.subcore_barrier()
        _fill_loop(resbuf, STRIPE, -jnp.inf)
        for t in range(NS):
            pltpu.sync_copy(smax_sh.at[t, pl.ds(sid * STRIPE, STRIPE)], tmpbuf)

            def red_body(i, _):
                sl = pl.ds(i * L, L)
                resbuf[sl] = jnp.maximum(resbuf[sl], tmpbuf[sl])
                return 0
            lax.fori_loop(0, STRIPE // L, red_body, 0)
        pltpu.sync_copy(resbuf, segmax_o.at[cid, pl.ds(sid * STRIPE, STRIPE)])

    return ka


def _make_kb(ep):
    ept = ep // NW
    ncht = ept // CH

    @functools.partial(
        pl.kernel,
        out_type=(jax.ShapeDtypeStruct((ep,), jnp.float32),
                  jax.ShapeDtypeStruct((NC, NP), jnp.float32)),
        mesh=_mesh,
        compiler_params=_params,
        scratch_types=[
            pltpu.VMEM((ept,), jnp.float32),
            pltpu.VMEM((ept,), jnp.int32),
            pltpu.VMEM((ept,), jnp.float32),
            pltpu.VMEM((NP,), jnp.float32),
            pltpu.VMEM((NP,), jnp.float32),
            pltpu.VMEM((CH,), jnp.int32),
            pltpu.VMEM((STRIPE,), jnp.float32),
            pltpu.VMEM_SHARED((NP,), jnp.float32),
        ],
    )
    def kb(scores, hseg, segmax, ex_o, segsum_o,
           score_sl, hseg_sl, ex_sl, gmax, mx2, seg_cp, stripe_v, ssum_sh):
        cid = lax.axis_index("c")
        sid = lax.axis_index("s")
        wid = cid * NS + sid
        base = wid * ept
        pltpu.sync_copy(scores.at[pl.ds(base, ept)], score_sl)
        pltpu.sync_copy(hseg.at[pl.ds(base, ept)], hseg_sl)
        pltpu.sync_copy(segmax.at[0], gmax)
        pltpu.sync_copy(segmax.at[1], mx2)

        def mb(i, _):
            sl = pl.ds(i * L, L)
            gmax[sl] = jnp.maximum(gmax[sl], mx2[sl])
            return 0
        lax.fori_loop(0, NP // L, mb, 0)

        _fill_loop(stripe_v, STRIPE, 0.0)
        pltpu.sync_copy(stripe_v, ssum_sh.at[pl.ds(sid * STRIPE, STRIPE)])
        plsc.subcore_barrier()

        def chunk_body(ci, _):
            def g_body(g, _):
                o = ci * CH + g * L
                sl = pl.ds(g * L, L)
                k16 = hseg_sl[pl.ds(o, L)]
                mx = plsc.load_gather(gmax, [k16])
                ex_sl[pl.ds(o, L)] = jnp.exp(score_sl[pl.ds(o, L)] - mx)
                seg_cp[sl] = k16
                return 0
            lax.fori_loop(0, CH // L, g_body, 0)
            pltpu.sync_copy(ex_sl.at[pl.ds(ci * CH, CH)],
                            ssum_sh.at[seg_cp], add=True)
            return 0
        lax.fori_loop(0, ncht, chunk_body, 0)
        pltpu.sync_copy(ex_sl, ex_o.at[pl.ds(base, ept)])

        plsc.subcore_barrier()
        pltpu.sync_copy(ssum_sh.at[pl.ds(sid * STRIPE, STRIPE)],
                        segsum_o.at[cid, pl.ds(sid * STRIPE, STRIPE)])

    return kb


def _make_kb2(ep):
    ept = ep // NW
    ncht = ept // CH

    @functools.partial(
        pl.kernel,
        out_type=jax.ShapeDtypeStruct((ep,), jnp.float32),
        mesh=_mesh,
        compiler_params=_params,
        scratch_types=[
            pltpu.VMEM((ept,), jnp.float32),
            pltpu.VMEM((ept,), jnp.int32),
            pltpu.VMEM((NP,), jnp.float32),
            pltpu.VMEM((NP,), jnp.float32),
        ],
    )
    def kb2(ex_in, seg_in, segsum, attn_o, ex_sl, seg_sl, ssum, s2):
        cid = lax.axis_index("c")
        sid = lax.axis_index("s")
        wid = cid * NS + sid
        base = wid * ept
        pltpu.sync_copy(ex_in.at[pl.ds(base, ept)], ex_sl)
        pltpu.sync_copy(seg_in.at[pl.ds(base, ept)], seg_sl)
        pltpu.sync_copy(segsum.at[0], ssum)
        pltpu.sync_copy(segsum.at[1], s2)

        def mb(i, _):
            sl = pl.ds(i * L, L)
            ssum[sl] = ssum[sl] + s2[sl]
            return 0
        lax.fori_loop(0, NP // L, mb, 0)

        def g_body(g, _):
            sl = pl.ds(g * L, L)
            ss = plsc.load_gather(ssum, [seg_sl[sl]])
            ex_sl[sl] = ex_sl[sl] / ss
            return 0
        lax.fori_loop(0, ept // L, g_body, 0)
        pltpu.sync_copy(ex_sl, attn_o.at[pl.ds(base, ept)])

    return kb2


def _make_agg(ep, n_out):
    ept = ep // NW
    ncht = ept // CH
    nb = ncht // BATCH
    chb = BATCH * CH
    tail_rows = n_out - (NS - 1) * STRIPE

    @functools.partial(
        pl.kernel,
        out_type=jax.ShapeDtypeStruct((NC, n_out, C), jnp.float32),
        mesh=_mesh,
        compiler_params=_params,
        scratch_types=[
            pltpu.VMEM((chb,), jnp.float32),
            pltpu.VMEM((chb,), jnp.int32),
            pltpu.VMEM((chb,), jnp.int32),
            pltpu.VMEM((CH, C), jnp.float32),
            pltpu.VMEM((CH, C), jnp.float32),
            pltpu.VMEM((CH,), jnp.int32),
            pltpu.VMEM_SHARED((NP, C), jnp.float32),
            pltpu.SemaphoreType.DMA,
            pltpu.SemaphoreType.DMA,
            pltpu.SemaphoreType.DMA,
        ],
    )
    def kagg(w_in, seg_in, gat_in, ent, part_o,
             w_b, seg_b, gat_b, rb0, rb1, seg_cp, accum, g0, g1, zs):
        cid = lax.axis_index("c")
        sid = lax.axis_index("s")
        wid = cid * NS + sid
        base = wid * ept
        rbufs, gsems = (rb0, rb1), (g0, g1)

        def zrow(e, _):
            for j in range(C // L):
                rb0[e, pl.ds(j * L, L)] = jnp.zeros((L,), jnp.float32)
            return 0
        lax.fori_loop(0, CH, zrow, 0)
        for k in range(STRIPE // CH):
            pltpu.async_copy(rb0, accum.at[pl.ds(sid * STRIPE + k * CH, CH), :],
                             zs)
        for k in range(STRIPE // CH):
            pltpu.make_async_copy(
                rb0, accum.at[pl.ds(sid * STRIPE + k * CH, CH), :], zs).wait()
        plsc.subcore_barrier()

        def issue(lc, b):
            pltpu.async_copy(ent.at[gat_b.at[pl.ds(lc * CH, CH)]],
                             rbufs[b], gsems[b])

        def batch_body(bi, _):
            boff = base + bi * chb
            pltpu.sync_copy(w_in.at[pl.ds(boff, chb)], w_b)
            pltpu.sync_copy(seg_in.at[pl.ds(boff, chb)], seg_b)
            pltpu.sync_copy(gat_in.at[pl.ds(boff, chb)], gat_b)
            issue(0, 0)
            issue(1, 1)

            def pair_body(p, _):
                for b in range(2):
                    lc = p * 2 + b
                    pltpu.make_async_copy(
                        ent.at[gat_b.at[pl.ds(lc * CH, CH)]],
                        rbufs[b], gsems[b]).wait()

                    def g_body(g, _):
                        sl = pl.ds(g * L, L)
                        o = lc * CH + g * L
                        seg_cp[sl] = seg_b[pl.ds(o, L)]
                        return 0
                    lax.fori_loop(0, CH // L, g_body, 0)
                    _scale_rows(rbufs[b], w_b.at[pl.ds(lc * CH, CH)])
                    pltpu.sync_copy(rbufs[b], accum.at[seg_cp], add=True)

                    @pl.when(lc + 2 < BATCH)
                    def _():
                        issue(lc + 2, b)
                return 0
            lax.fori_loop(0, BATCH // 2, pair_body, 0)
            return 0
        lax.fori_loop(0, nb, batch_body, 0)

        plsc.subcore_barrier()

        @pl.when(sid < NS - 1)
        def _():
            pltpu.sync_copy(accum.at[pl.ds(sid * STRIPE, STRIPE), :],
                            part_o.at[cid, pl.ds(sid * STRIPE, STRIPE), :])

        @pl.when(sid == NS - 1)
        def _():
            pltpu.sync_copy(
                accum.at[pl.ds((NS - 1) * STRIPE, tail_rows), :],
                part_o.at[cid, pl.ds((NS - 1) * STRIPE, tail_rows), :])

    return kagg


def _norm_combine(parts):
    n = parts.shape[1]
    br = 400
    assert n % br == 0

    def body(p_ref, o_ref):
        s = p_ref[0] + p_ref[1]
        ss = jnp.sum(s * s, axis=1, keepdims=True)
        nrm = jnp.sqrt(ss)
        o_ref[...] = s / jnp.maximum(nrm, 1e-12)

    return pl.pallas_call(
        body,
        grid=(n // br,),
        in_specs=[pl.BlockSpec((NC, br, C), lambda i: (0, i, 0))],
        out_specs=pl.BlockSpec((br, C), lambda i: (i, 0)),
        out_shape=jax.ShapeDtypeStruct((n, C), jnp.float32),
    )(parts)


def kernel(user_emb, item_emb, edge_index, edge_type, inter_edge,
           inter_edge_w, relation_emb):
    e = edge_index.shape[1]
    gran = NW * CH * BATCH
    ep = ((e + gran - 1) // gran) * gran
    ni = inter_edge.shape[1]
    nip = ((ni + gran - 1) // gran) * gran

    head = edge_index[0]
    tail = edge_index[1]
    hseg = _pad_to(head, ep, N_ENT)
    hgat = _pad_to(head, ep, 0)
    tgat = _pad_to(tail, ep, 0)
    ridx = _pad_to((edge_type + NR - 2) % (NR - 1), ep, 0)

    dseg = _pad_to(inter_edge[0], nip, N_USR)
    sgat = _pad_to(inter_edge[1], nip, 0)
    w_i = _pad_to(inter_edge_w, nip, 0.0)

    ka = _make_ka(ep)
    kb = _make_kb(ep)
    kb2 = _make_kb2(ep)
    kc = _make_agg(ep, N_ENT)
    ke = _make_agg(nip, N_USR)

    entity = item_emb
    for _ in range(2):
        scores, segmax = ka(entity, hgat, hseg, tgat, ridx, relation_emb)
        ex, segsum = kb(scores, hseg, segmax)
        attn = kb2(ex, hseg, segsum)
        parts = kc(attn, hseg, tgat, entity)
        entity = _norm_combine(parts)

    uparts = ke(w_i, dseg, sgat, entity)
    user_out = _norm_combine(uparts)
    return (user_out, entity)


# speedup vs baseline: 1.6412x; 1.0381x over previous
import functools

import jax
import jax.numpy as jnp
from jax import lax
from jax.experimental import pallas as pl
from jax.experimental.pallas import tpu as pltpu
from jax.experimental.pallas import tpu_sc as plsc

NC, NS, L = 2, 16, 16
NW = NC * NS
N_ENT = 10000
N_USR = 10000
C = 128
NR = 12
NP = 10240
STRIPE = NP // NS
CH = 128
BATCH = 16

_mesh = plsc.VectorSubcoreMesh(core_axis_name="c", subcore_axis_name="s")
_params = pltpu.CompilerParams(needs_layout_passes=False)


def _pad_to(x, n, fill):
    return jnp.concatenate(
        [x, jnp.full((n - x.shape[0],), fill, x.dtype)]) if x.shape[0] != n else x


def _fill_loop(ref, n, value):
    def body(i, _):
        ref[pl.ds(i * L, L)] = jnp.full((L,), value, jnp.float32)
        return 0
    lax.fori_loop(0, n // L, body, 0, unroll=8)


def _scale_rows(rows, w_v):
    iota = lax.iota(jnp.int32, L)

    def grp(g, _):
        sl = pl.ds(g * L, L)
        w16 = w_v[sl]
        e16 = iota + g * L

        def c_body(c, _):
            c16 = jnp.full((L,), 0, jnp.int32) + c
            v = plsc.load_gather(rows, [e16, c16])
            plsc.store_scatter(rows, [e16, c16], v * w16)
            return 0
        lax.fori_loop(0, C, c_body, 0, unroll=8)
        return 0
    lax.fori_loop(0, CH // L, grp, 0)


def _seg_run_max(k16, v16):
    ks, vs = plsc.sort_key_val(k16, v16)
    iota = lax.iota(jnp.int32, L)
    for d in (1, 2, 4, 8):
        idx = jnp.maximum(iota - d, 0)
        kp = ks.at[idx].get(mode="promise_in_bounds")
        vp = vs.at[idx].get(mode="promise_in_bounds")
        same = (kp == ks) & (iota >= d)
        vs = jnp.where(same, jnp.maximum(vs, vp), vs)
    nxt = jnp.minimum(iota + 1, L - 1)
    kn = ks.at[nxt].get(mode="promise_in_bounds")
    is_last = (kn != ks) | (iota == L - 1)
    return ks, vs, is_last


def _make_ka(ep):
    ept = ep // NW
    ncht = ept // CH
    npairs = (ncht + 1) // 2

    @functools.partial(
        pl.kernel,
        out_type=(jax.ShapeDtypeStruct((ep,), jnp.float32),
                  jax.ShapeDtypeStruct((NC, NP), jnp.float32)),
        mesh=_mesh,
        compiler_params=_params,
        scratch_types=[
            pltpu.VMEM((ept,), jnp.int32),
            pltpu.VMEM((ept,), jnp.int32),
            pltpu.VMEM((ept,), jnp.int32),
            pltpu.VMEM((ept,), jnp.int32),
            pltpu.VMEM((NR - 1, C), jnp.float32),
            pltpu.VMEM((CH, C), jnp.float32),
            pltpu.VMEM((CH, C), jnp.float32),
            pltpu.VMEM((CH, C), jnp.float32),
            pltpu.VMEM((CH, C), jnp.float32),
            pltpu.VMEM((CH,), jnp.float32),
            pltpu.VMEM((NP,), jnp.float32),
            pltpu.VMEM_SHARED((NS, NP), jnp.float32),
            pltpu.VMEM((STRIPE,), jnp.float32),
            pltpu.VMEM((STRIPE,), jnp.float32),
            pltpu.SemaphoreType.DMA,
            pltpu.SemaphoreType.DMA,
            pltpu.SemaphoreType.DMA,
            pltpu.SemaphoreType.DMA,
        ],
    )
    def ka(ent, hgat, hseg, tgat, ridx, rel, scores_o, segmax_o,
           hgat_sl, hseg_sl, tgat_sl, ridx_sl, relv, hb0, hb1, tb0, tb1,
           score_v, maxarr, smax_sh, tmpbuf, resbuf, hs0, hs1, ts0, ts1):
        cid = lax.axis_index("c")
        sid = lax.axis_index("s")
        wid = cid * NS + sid
        base = wid * ept
        hbufs, tbufs = (hb0, hb1), (tb0, tb1)
        hsems, tsems = (hs0, hs1), (ts0, ts1)

        pltpu.sync_copy(rel, relv)
        pltpu.sync_copy(hgat.at[pl.ds(base, ept)], hgat_sl)
        pltpu.sync_copy(hseg.at[pl.ds(base, ept)], hseg_sl)
        pltpu.sync_copy(tgat.at[pl.ds(base, ept)], tgat_sl)
        pltpu.sync_copy(ridx.at[pl.ds(base, ept)], ridx_sl)
        _fill_loop(maxarr, NP, -jnp.inf)

        def issue(ci, b):
            pltpu.async_copy(ent.at[hgat_sl.at[pl.ds(ci * CH, CH)]],
                             hbufs[b], hsems[b])
            pltpu.async_copy(ent.at[tgat_sl.at[pl.ds(ci * CH, CH)]],
                             tbufs[b], tsems[b])

        issue(0, 0)
        issue(1, 1)
        iota = lax.iota(jnp.int32, L)

        def pair_body(p, _):
            for b in range(2):
                ci = p * 2 + b

                @pl.when(ci < ncht)
                def _():
                    pltpu.make_async_copy(
                        ent.at[hgat_sl.at[pl.ds(ci * CH, CH)]],
                        hbufs[b], hsems[b]).wait()
                    pltpu.make_async_copy(
                        ent.at[tgat_sl.at[pl.ds(ci * CH, CH)]],
                        tbufs[b], tsems[b]).wait()

                    def grp_scores(g, _):
                        sl = pl.ds(g * L, L)
                        e16 = iota + g * L
                        r16 = ridx_sl[pl.ds(ci * CH + g * L, L)]

                        def c_body(c, acc):
                            c16 = jnp.full((L,), 0, jnp.int32) + c
                            h = plsc.load_gather(hbufs[b], [e16, c16])
                            t = plsc.load_gather(tbufs[b], [e16, c16])
                            rv = plsc.load_gather(relv, [r16, c16])
                            return acc + h * t * rv
                        dot = lax.fori_loop(0, C, c_body,
                                            jnp.zeros((L,), jnp.float32),
                                            unroll=8)
                        score_v[sl] = jnp.exp(dot)
                        return 0
                    lax.fori_loop(0, CH // L, grp_scores, 0)
                    pltpu.sync_copy(score_v,
                                    scores_o.at[pl.ds(base + ci * CH, CH)])

                    def g_body(g, _):
                        sl = pl.ds(g * L, L)
                        k16 = hseg_sl[pl.ds(ci * CH + g * L, L)]
                        ks, vs, is_last = _seg_run_max(k16, score_v[sl])
                        cur = plsc.load_gather(maxarr, [ks], mask=is_last)
                        plsc.store_scatter(maxarr, [ks],
                                           jnp.maximum(vs, cur), mask=is_last)
                        return 0
                    lax.fori_loop(0, CH // L, g_body, 0)

                    @pl.when(ci + 2 < ncht)
                    def _():
                        issue(ci + 2, b)
            return 0
        lax.fori_loop(0, npairs, pair_body, 0)

        pltpu.sync_copy(maxarr, smax_sh.at[sid])
        plsc.subcore_barrier()
        _fill_loop(resbuf, STRIPE, -jnp.inf)
        for t in range(NS):
            pltpu.sync_copy(smax_sh.at[t, pl.ds(sid * STRIPE, STRIPE)], tmpbuf)

            def red_body(i, _):
                sl = pl.ds(i * L, L)
                resbuf[sl] = jnp.maximum(resbuf[sl], tmpbuf[sl])
                return 0
            lax.fori_loop(0, STRIPE // L, red_body, 0, unroll=8)
        pltpu.sync_copy(resbuf, segmax_o.at[cid, pl.ds(sid * STRIPE, STRIPE)])

    return ka


def _make_kb(ep):
    ept = ep // NW
    ncht = ept // CH

    @functools.partial(
        pl.kernel,
        out_type=(jax.ShapeDtypeStruct((ep,), jnp.float32),
                  jax.ShapeDtypeStruct((NC, NP), jnp.float32)),
        mesh=_mesh,
        compiler_params=_params,
        scratch_types=[
            pltpu.VMEM((ept,), jnp.float32),
            pltpu.VMEM((ept,), jnp.int32),
            pltpu.VMEM((ept,), jnp.float32),
            pltpu.VMEM((NP,), jnp.float32),
            pltpu.VMEM((NP,), jnp.float32),
            pltpu.VMEM((CH,), jnp.int32),
            pltpu.VMEM((STRIPE,), jnp.float32),
            pltpu.VMEM_SHARED((NP,), jnp.float32),
        ],
    )
    def kb(scores, hseg, segmax, ex_o, segsum_o,
           score_sl, hseg_sl, ex_sl, gmax, mx2, seg_cp, stripe_v, ssum_sh):
        cid = lax.axis_index("c")
        sid = lax.axis_index("s")
        wid = cid * NS + sid
        base = wid * ept
        pltpu.sync_copy(scores.at[pl.ds(base, ept)], score_sl)
        pltpu.sync_copy(hseg.at[pl.ds(base, ept)], hseg_sl)
        pltpu.sync_copy(segmax.at[0], gmax)
        pltpu.sync_copy(segmax.at[1], mx2)

        def mb(i, _):
            sl = pl.ds(i * L, L)
            gmax[sl] = jnp.maximum(gmax[sl], mx2[sl])
            return 0
        lax.fori_loop(0, NP // L, mb, 0, unroll=8)

        _fill_loop(stripe_v, STRIPE, 0.0)
        pltpu.sync_copy(stripe_v, ssum_sh.at[pl.ds(sid * STRIPE, STRIPE)])
        plsc.subcore_barrier()

        def chunk_body(ci, _):
            def g_body(g, _):
                o = ci * CH + g * L
                sl = pl.ds(g * L, L)
                k16 = hseg_sl[pl.ds(o, L)]
                mx = plsc.load_gather(gmax, [k16])
                ex_sl[pl.ds(o, L)] = jnp.exp(score_sl[pl.ds(o, L)] - mx)
                seg_cp[sl] = k16
                return 0
            lax.fori_loop(0, CH // L, g_body, 0, unroll=4)
            pltpu.sync_copy(ex_sl.at[pl.ds(ci * CH, CH)],
                            ssum_sh.at[seg_cp], add=True)
            return 0
        lax.fori_loop(0, ncht, chunk_body, 0)
        pltpu.sync_copy(ex_sl, ex_o.at[pl.ds(base, ept)])

        plsc.subcore_barrier()
        pltpu.sync_copy(ssum_sh.at[pl.ds(sid * STRIPE, STRIPE)],
                        segsum_o.at[cid, pl.ds(sid * STRIPE, STRIPE)])

    return kb


def _make_kb2(ep):
    ept = ep // NW
    ncht = ept // CH

    @functools.partial(
        pl.kernel,
        out_type=jax.ShapeDtypeStruct((ep,), jnp.float32),
        mesh=_mesh,
        compiler_params=_params,
        scratch_types=[
            pltpu.VMEM((ept,), jnp.float32),
            pltpu.VMEM((ept,), jnp.int32),
            pltpu.VMEM((NP,), jnp.float32),
            pltpu.VMEM((NP,), jnp.float32),
        ],
    )
    def kb2(ex_in, seg_in, segsum, attn_o, ex_sl, seg_sl, ssum, s2):
        cid = lax.axis_index("c")
        sid = lax.axis_index("s")
        wid = cid * NS + sid
        base = wid * ept
        pltpu.sync_copy(ex_in.at[pl.ds(base, ept)], ex_sl)
        pltpu.sync_copy(seg_in.at[pl.ds(base, ept)], seg_sl)
        pltpu.sync_copy(segsum.at[0], ssum)
        pltpu.sync_copy(segsum.at[1], s2)

        def mb(i, _):
            sl = pl.ds(i * L, L)
            ssum[sl] = ssum[sl] + s2[sl]
            return 0
        lax.fori_loop(0, NP // L, mb, 0, unroll=8)

        def g_body(g, _):
            sl = pl.ds(g * L, L)
            ss = plsc.load_gather(ssum, [seg_sl[sl]])
            ex_sl[sl] = ex_sl[sl] / ss
            return 0
        lax.fori_loop(0, ept // L, g_body, 0, unroll=8)
        pltpu.sync_copy(ex_sl, attn_o.at[pl.ds(base, ept)])

    return kb2


def _make_agg(ep, n_out):
    ept = ep // NW
    ncht = ept // CH
    nb = ncht // BATCH
    chb = BATCH * CH
    tail_rows = n_out - (NS - 1) * STRIPE

    @functools.partial(
        pl.kernel,
        out_type=jax.ShapeDtypeStruct((NC, n_out, C), jnp.float32),
        mesh=_mesh,
        compiler_params=_params,
        scratch_types=[
            pltpu.VMEM((chb,), jnp.float32),
            pltpu.VMEM((chb,), jnp.int32),
            pltpu.VMEM((chb,), jnp.int32),
            pltpu.VMEM((CH, C), jnp.float32),
            pltpu.VMEM((CH, C), jnp.float32),
            pltpu.VMEM((CH,), jnp.int32),
            pltpu.VMEM_SHARED((NP, C), jnp.float32),
            pltpu.SemaphoreType.DMA,
            pltpu.SemaphoreType.DMA,
            pltpu.SemaphoreType.DMA,
        ],
    )
    def kagg(w_in, seg_in, gat_in, ent, part_o,
             w_b, seg_b, gat_b, rb0, rb1, seg_cp, accum, g0, g1, zs):
        cid = lax.axis_index("c")
        sid = lax.axis_index("s")
        wid = cid * NS + sid
        base = wid * ept
        rbufs, gsems = (rb0, rb1), (g0, g1)

        def zrow(e, _):
            for j in range(C // L):
                rb0[e, pl.ds(j * L, L)] = jnp.zeros((L,), jnp.float32)
            return 0
        lax.fori_loop(0, CH, zrow, 0, unroll=4)
        for k in range(STRIPE // CH):
            pltpu.async_copy(rb0, accum.at[pl.ds(sid * STRIPE + k * CH, CH), :],
                             zs)
        for k in range(STRIPE // CH):
            pltpu.make_async_copy(
                rb0, accum.at[pl.ds(sid * STRIPE + k * CH, CH), :], zs).wait()
        plsc.subcore_barrier()

        def issue(lc, b):
            pltpu.async_copy(ent.at[gat_b.at[pl.ds(lc * CH, CH)]],
                             rbufs[b], gsems[b])

        def batch_body(bi, _):
            boff = base + bi * chb
            pltpu.sync_copy(w_in.at[pl.ds(boff, chb)], w_b)
            pltpu.sync_copy(seg_in.at[pl.ds(boff, chb)], seg_b)
            pltpu.sync_copy(gat_in.at[pl.ds(boff, chb)], gat_b)
            issue(0, 0)
            issue(1, 1)

            def pair_body(p, _):
                for b in range(2):
                    lc = p * 2 + b
                    pltpu.make_async_copy(
                        ent.at[gat_b.at[pl.ds(lc * CH, CH)]],
                        rbufs[b], gsems[b]).wait()

                    def g_body(g, _):
                        sl = pl.ds(g * L, L)
                        o = lc * CH + g * L
                        seg_cp[sl] = seg_b[pl.ds(o, L)]
                        return 0
                    lax.fori_loop(0, CH // L, g_body, 0, unroll=4)
                    _scale_rows(rbufs[b], w_b.at[pl.ds(lc * CH, CH)])
                    pltpu.sync_copy(rbufs[b], accum.at[seg_cp], add=True)

                    @pl.when(lc + 2 < BATCH)
                    def _():
                        issue(lc + 2, b)
                return 0
            lax.fori_loop(0, BATCH // 2, pair_body, 0)
            return 0
        lax.fori_loop(0, nb, batch_body, 0)

        plsc.subcore_barrier()

        @pl.when(sid < NS - 1)
        def _():
            pltpu.sync_copy(accum.at[pl.ds(sid * STRIPE, STRIPE), :],
                            part_o.at[cid, pl.ds(sid * STRIPE, STRIPE), :])

        @pl.when(sid == NS - 1)
        def _():
            pltpu.sync_copy(
                accum.at[pl.ds((NS - 1) * STRIPE, tail_rows), :],
                part_o.at[cid, pl.ds((NS - 1) * STRIPE, tail_rows), :])

    return kagg


def _norm_combine(parts):
    n = parts.shape[1]
    br = 400
    assert n % br == 0

    def body(p_ref, o_ref):
        s = p_ref[0] + p_ref[1]
        ss = jnp.sum(s * s, axis=1, keepdims=True)
        nrm = jnp.sqrt(ss)
        o_ref[...] = s / jnp.maximum(nrm, 1e-12)

    return pl.pallas_call(
        body,
        grid=(n // br,),
        in_specs=[pl.BlockSpec((NC, br, C), lambda i: (0, i, 0))],
        out_specs=pl.BlockSpec((br, C), lambda i: (i, 0)),
        out_shape=jax.ShapeDtypeStruct((n, C), jnp.float32),
    )(parts)


def kernel(user_emb, item_emb, edge_index, edge_type, inter_edge,
           inter_edge_w, relation_emb):
    e = edge_index.shape[1]
    gran = NW * CH * BATCH
    ep = ((e + gran - 1) // gran) * gran
    ni = inter_edge.shape[1]
    nip = ((ni + gran - 1) // gran) * gran

    head = edge_index[0]
    tail = edge_index[1]
    hseg = _pad_to(head, ep, N_ENT)
    hgat = _pad_to(head, ep, 0)
    tgat = _pad_to(tail, ep, 0)
    ridx = _pad_to((edge_type + NR - 2) % (NR - 1), ep, 0)

    dseg = _pad_to(inter_edge[0], nip, N_USR)
    sgat = _pad_to(inter_edge[1], nip, 0)
    w_i = _pad_to(inter_edge_w, nip, 0.0)

    ka = _make_ka(ep)
    kb = _make_kb(ep)
    kb2 = _make_kb2(ep)
    kc = _make_agg(ep, N_ENT)
    ke = _make_agg(nip, N_USR)

    entity = item_emb
    for _ in range(2):
        scores, segmax = ka(entity, hgat, hseg, tgat, ridx, relation_emb)
        ex, segsum = kb(scores, hseg, segmax)
        attn = kb2(ex, hseg, segsum)
        parts = kc(attn, hseg, tgat, entity)
        entity = _norm_combine(parts)

    uparts = ke(w_i, dseg, sgat, entity)
    user_out = _norm_combine(uparts)
    return (user_out, entity)


# speedup vs baseline: 2.5881x; 1.5770x over previous
import functools

import jax
import jax.numpy as jnp
from jax import lax
from jax.experimental import pallas as pl
from jax.experimental.pallas import tpu as pltpu
from jax.experimental.pallas import tpu_sc as plsc

NC, NS, L = 2, 16, 16
NW = NC * NS
N_ENT = 10000
N_USR = 10000
C = 128
NR = 12
NP = 10240
STRIPE = NP // NS
CH = 128
BATCH = 16

_mesh = plsc.VectorSubcoreMesh(core_axis_name="c", subcore_axis_name="s")
_params = pltpu.CompilerParams(needs_layout_passes=False)


def _pad_to(x, n, fill):
    return jnp.concatenate(
        [x, jnp.full((n - x.shape[0],), fill, x.dtype)]) if x.shape[0] != n else x


def _fill_loop(ref, n, value):
    def body(i, _):
        ref[pl.ds(i * L, L)] = jnp.full((L,), value, jnp.float32)
        return 0
    lax.fori_loop(0, n // L, body, 0, unroll=8)


def _scale_rows(rows, w_v):
    iota = lax.iota(jnp.int32, L)

    def grp(g, _):
        w16 = w_v[pl.ds(g * L, L)]
        for i in range(L):
            e = g * L + i
            bc = w16.at[jnp.full((L,), i, jnp.int32)].get(
                mode="promise_in_bounds")
            for j in range(C // L):
                sl = pl.ds(j * L, L)
                rows[e, sl] = rows[e, sl] * bc
        return 0
    lax.fori_loop(0, CH // L, grp, 0)


def _seg_run_max(k16, v16):
    ks, vs = plsc.sort_key_val(k16, v16)
    iota = lax.iota(jnp.int32, L)
    for d in (1, 2, 4, 8):
        idx = jnp.maximum(iota - d, 0)
        kp = ks.at[idx].get(mode="promise_in_bounds")
        vp = vs.at[idx].get(mode="promise_in_bounds")
        same = (kp == ks) & (iota >= d)
        vs = jnp.where(same, jnp.maximum(vs, vp), vs)
    nxt = jnp.minimum(iota + 1, L - 1)
    kn = ks.at[nxt].get(mode="promise_in_bounds")
    is_last = (kn != ks) | (iota == L - 1)
    return ks, vs, is_last


def _make_ka(ep):
    ept = ep // NW
    ncht = ept // CH

    @functools.partial(
        pl.kernel,
        out_type=(jax.ShapeDtypeStruct((ep,), jnp.float32),
                  jax.ShapeDtypeStruct((NC, NP), jnp.float32)),
        mesh=_mesh,
        compiler_params=_params,
        scratch_types=[
            pltpu.VMEM((ept,), jnp.int32),
            pltpu.VMEM((ept,), jnp.int32),
            pltpu.VMEM((ept,), jnp.int32),
            pltpu.VMEM((ept,), jnp.int32),
            pltpu.VMEM((CH, C), jnp.float32),
            pltpu.VMEM((CH, C), jnp.float32),
            pltpu.VMEM((CH, C), jnp.float32),
            pltpu.VMEM((CH,), jnp.float32),
            pltpu.VMEM((NP,), jnp.float32),
            pltpu.VMEM_SHARED((NS, NP), jnp.float32),
            pltpu.VMEM((STRIPE,), jnp.float32),
            pltpu.VMEM((STRIPE,), jnp.float32),
        ],
    )
    def ka(ent, hgat, hseg, tgat, ridx, relrep, scores_o, segmax_o,
           hgat_sl, hseg_sl, tgat_sl, ridx_sl, hrow, trow, rrow,
           score_v, maxarr, smax_sh, tmpbuf, resbuf):
        cid = lax.axis_index("c")
        sid = lax.axis_index("s")
        wid = cid * NS + sid
        base = wid * ept

        pltpu.sync_copy(hgat.at[pl.ds(base, ept)], hgat_sl)
        pltpu.sync_copy(hseg.at[pl.ds(base, ept)], hseg_sl)
        pltpu.sync_copy(tgat.at[pl.ds(base, ept)], tgat_sl)
        pltpu.sync_copy(ridx.at[pl.ds(base, ept)], ridx_sl)
        _fill_loop(maxarr, NP, -jnp.inf)
        iota = lax.iota(jnp.int32, L)

        def chunk_body(ci, _):
            pltpu.sync_copy(ent.at[hgat_sl.at[pl.ds(ci * CH, CH)]], hrow)
            pltpu.sync_copy(ent.at[tgat_sl.at[pl.ds(ci * CH, CH)]], trow)
            pltpu.sync_copy(relrep.at[ridx_sl.at[pl.ds(ci * CH, CH)]], rrow)

            def grp_scores(g, _):
                svec = jnp.zeros((L,), jnp.float32)
                for i in range(L):
                    e = g * L + i
                    acc = jnp.zeros((L,), jnp.float32)
                    for j in range(C // L):
                        slj = pl.ds(j * L, L)
                        acc = acc + hrow[e, slj] * trow[e, slj] * rrow[e, slj]
                    for d in (1, 2, 4, 8):
                        acc = acc + acc.at[iota ^ d].get(
                            mode="promise_in_bounds")
                    svec = jnp.where(iota == i, acc, svec)
                score_v[pl.ds(g * L, L)] = jnp.exp(svec)
                return 0
            lax.fori_loop(0, CH // L, grp_scores, 0)
            pltpu.sync_copy(score_v, scores_o.at[pl.ds(base + ci * CH, CH)])

            def g_body(g, _):
                sl = pl.ds(g * L, L)
                k16 = hseg_sl[pl.ds(ci * CH + g * L, L)]
                ks, vs, is_last = _seg_run_max(k16, score_v[sl])
                cur = plsc.load_gather(maxarr, [ks], mask=is_last)
                plsc.store_scatter(maxarr, [ks],
                                   jnp.maximum(vs, cur), mask=is_last)
                return 0
            lax.fori_loop(0, CH // L, g_body, 0)
            return 0
        lax.fori_loop(0, ncht, chunk_body, 0)

        pltpu.sync_copy(maxarr, smax_sh.at[sid])
        plsc.subcore_barrier()
        _fill_loop(resbuf, STRIPE, -jnp.inf)
        for t in range(NS):
            pltpu.sync_copy(smax_sh.at[t, pl.ds(sid * STRIPE, STRIPE)], tmpbuf)

            def red_body(i, _):
                sl = pl.ds(i * L, L)
                resbuf[sl] = jnp.maximum(resbuf[sl], tmpbuf[sl])
                return 0
            lax.fori_loop(0, STRIPE // L, red_body, 0, unroll=8)
        pltpu.sync_copy(resbuf, segmax_o.at[cid, pl.ds(sid * STRIPE, STRIPE)])

    return ka


def _make_kb(ep):
    ept = ep // NW
    ncht = ept // CH

    @functools.partial(
        pl.kernel,
        out_type=(jax.ShapeDtypeStruct((ep,), jnp.float32),
                  jax.ShapeDtypeStruct((NC, NP), jnp.float32)),
        mesh=_mesh,
        compiler_params=_params,
        scratch_types=[
            pltpu.VMEM((ept,), jnp.float32),
            pltpu.VMEM((ept,), jnp.int32),
            pltpu.VMEM((ept,), jnp.float32),
            pltpu.VMEM((NP,), jnp.float32),
            pltpu.VMEM((NP,), jnp.float32),
            pltpu.VMEM((CH,), jnp.int32),
            pltpu.VMEM((STRIPE,), jnp.float32),
            pltpu.VMEM_SHARED((NP,), jnp.float32),
        ],
    )
    def kb(scores, hseg, segmax, ex_o, segsum_o,
           score_sl, hseg_sl, ex_sl, gmax, mx2, seg_cp, stripe_v, ssum_sh):
        cid = lax.axis_index("c")
        sid = lax.axis_index("s")
        wid = cid * NS + sid
        base = wid * ept
        pltpu.sync_copy(scores.at[pl.ds(base, ept)], score_sl)
        pltpu.sync_copy(hseg.at[pl.ds(base, ept)], hseg_sl)
        pltpu.sync_copy(segmax.at[0], gmax)
        pltpu.sync_copy(segmax.at[1], mx2)

        def mb(i, _):
            sl = pl.ds(i * L, L)
            gmax[sl] = jnp.maximum(gmax[sl], mx2[sl])
            return 0
        lax.fori_loop(0, NP // L, mb, 0, unroll=8)

        _fill_loop(stripe_v, STRIPE, 0.0)
        pltpu.sync_copy(stripe_v, ssum_sh.at[pl.ds(sid * STRIPE, STRIPE)])
        plsc.subcore_barrier()

        def chunk_body(ci, _):
            def g_body(g, _):
                o = ci * CH + g * L
                sl = pl.ds(g * L, L)
                k16 = hseg_sl[pl.ds(o, L)]
                mx = plsc.load_gather(gmax, [k16])
                ex_sl[pl.ds(o, L)] = jnp.exp(score_sl[pl.ds(o, L)] - mx)
                seg_cp[sl] = k16
                return 0
            lax.fori_loop(0, CH // L, g_body, 0, unroll=4)
            pltpu.sync_copy(ex_sl.at[pl.ds(ci * CH, CH)],
                            ssum_sh.at[seg_cp], add=True)
            return 0
        lax.fori_loop(0, ncht, chunk_body, 0)
        pltpu.sync_copy(ex_sl, ex_o.at[pl.ds(base, ept)])

        plsc.subcore_barrier()
        pltpu.sync_copy(ssum_sh.at[pl.ds(sid * STRIPE, STRIPE)],
                        segsum_o.at[cid, pl.ds(sid * STRIPE, STRIPE)])

    return kb


def _make_kb2(ep):
    ept = ep // NW
    ncht = ept // CH

    @functools.partial(
        pl.kernel,
        out_type=jax.ShapeDtypeStruct((ep,), jnp.float32),
        mesh=_mesh,
        compiler_params=_params,
        scratch_types=[
            pltpu.VMEM((ept,), jnp.float32),
            pltpu.VMEM((ept,), jnp.int32),
            pltpu.VMEM((NP,), jnp.float32),
            pltpu.VMEM((NP,), jnp.float32),
        ],
    )
    def kb2(ex_in, seg_in, segsum, attn_o, ex_sl, seg_sl, ssum, s2):
        cid = lax.axis_index("c")
        sid = lax.axis_index("s")
        wid = cid * NS + sid
        base = wid * ept
        pltpu.sync_copy(ex_in.at[pl.ds(base, ept)], ex_sl)
        pltpu.sync_copy(seg_in.at[pl.ds(base, ept)], seg_sl)
        pltpu.sync_copy(segsum.at[0], ssum)
        pltpu.sync_copy(segsum.at[1], s2)

        def mb(i, _):
            sl = pl.ds(i * L, L)
            ssum[sl] = ssum[sl] + s2[sl]
            return 0
        lax.fori_loop(0, NP // L, mb, 0, unroll=8)

        def g_body(g, _):
            sl = pl.ds(g * L, L)
            ss = plsc.load_gather(ssum, [seg_sl[sl]])
            ex_sl[sl] = ex_sl[sl] / ss
            return 0
        lax.fori_loop(0, ept // L, g_body, 0, unroll=8)
        pltpu.sync_copy(ex_sl, attn_o.at[pl.ds(base, ept)])

    return kb2


def _make_agg(ep, n_out):
    ept = ep // NW
    ncht = ept // CH
    nb = ncht // BATCH
    chb = BATCH * CH
    tail_rows = n_out - (NS - 1) * STRIPE

    @functools.partial(
        pl.kernel,
        out_type=jax.ShapeDtypeStruct((NC, n_out, C), jnp.float32),
        mesh=_mesh,
        compiler_params=_params,
        scratch_types=[
            pltpu.VMEM((chb,), jnp.float32),
            pltpu.VMEM((chb,), jnp.int32),
            pltpu.VMEM((chb,), jnp.int32),
            pltpu.VMEM((CH, C), jnp.float32),
            pltpu.VMEM((CH, C), jnp.float32),
            pltpu.VMEM((CH,), jnp.int32),
            pltpu.VMEM_SHARED((NP, C), jnp.float32),
            pltpu.SemaphoreType.DMA,
            pltpu.SemaphoreType.DMA,
            pltpu.SemaphoreType.DMA,
        ],
    )
    def kagg(w_in, seg_in, gat_in, ent, part_o,
             w_b, seg_b, gat_b, rb0, rb1, seg_cp, accum, g0, g1, zs):
        cid = lax.axis_index("c")
        sid = lax.axis_index("s")
        wid = cid * NS + sid
        base = wid * ept
        rbufs, gsems = (rb0, rb1), (g0, g1)

        def zrow(e, _):
            for j in range(C // L):
                rb0[e, pl.ds(j * L, L)] = jnp.zeros((L,), jnp.float32)
            return 0
        lax.fori_loop(0, CH, zrow, 0, unroll=4)
        for k in range(STRIPE // CH):
            pltpu.async_copy(rb0, accum.at[pl.ds(sid * STRIPE + k * CH, CH), :],
                             zs)
        for k in range(STRIPE // CH):
            pltpu.make_async_copy(
                rb0, accum.at[pl.ds(sid * STRIPE + k * CH, CH), :], zs).wait()
        plsc.subcore_barrier()

        def issue(lc, b):
            pltpu.async_copy(ent.at[gat_b.at[pl.ds(lc * CH, CH)]],
                             rbufs[b], gsems[b])

        def batch_body(bi, _):
            boff = base + bi * chb
            pltpu.sync_copy(w_in.at[pl.ds(boff, chb)], w_b)
            pltpu.sync_copy(seg_in.at[pl.ds(boff, chb)], seg_b)
            pltpu.sync_copy(gat_in.at[pl.ds(boff, chb)], gat_b)
            issue(0, 0)
            issue(1, 1)

            def pair_body(p, _):
                for b in range(2):
                    lc = p * 2 + b
                    pltpu.make_async_copy(
                        ent.at[gat_b.at[pl.ds(lc * CH, CH)]],
                        rbufs[b], gsems[b]).wait()

                    def g_body(g, _):
                        sl = pl.ds(g * L, L)
                        o = lc * CH + g * L
                        seg_cp[sl] = seg_b[pl.ds(o, L)]
                        return 0
                    lax.fori_loop(0, CH // L, g_body, 0, unroll=4)
                    _scale_rows(rbufs[b], w_b.at[pl.ds(lc * CH, CH)])
                    pltpu.sync_copy(rbufs[b], accum.at[seg_cp], add=True)

                    @pl.when(lc + 2 < BATCH)
                    def _():
                        issue(lc + 2, b)
                return 0
            lax.fori_loop(0, BATCH // 2, pair_body, 0)
            return 0
        lax.fori_loop(0, nb, batch_body, 0)

        plsc.subcore_barrier()

        @pl.when(sid < NS - 1)
        def _():
            pltpu.sync_copy(accum.at[pl.ds(sid * STRIPE, STRIPE), :],
                            part_o.at[cid, pl.ds(sid * STRIPE, STRIPE), :])

        @pl.when(sid == NS - 1)
        def _():
            pltpu.sync_copy(
                accum.at[pl.ds((NS - 1) * STRIPE, tail_rows), :],
                part_o.at[cid, pl.ds((NS - 1) * STRIPE, tail_rows), :])

    return kagg


def _norm_combine(parts):
    n = parts.shape[1]
    br = 400
    assert n % br == 0

    def body(p_ref, o_ref):
        s = p_ref[0] + p_ref[1]
        ss = jnp.sum(s * s, axis=1, keepdims=True)
        nrm = jnp.sqrt(ss)
        o_ref[...] = s / jnp.maximum(nrm, 1e-12)

    return pl.pallas_call(
        body,
        grid=(n // br,),
        in_specs=[pl.BlockSpec((NC, br, C), lambda i: (0, i, 0))],
        out_specs=pl.BlockSpec((br, C), lambda i: (i, 0)),
        out_shape=jax.ShapeDtypeStruct((n, C), jnp.float32),
    )(parts)


def kernel(user_emb, item_emb, edge_index, edge_type, inter_edge,
           inter_edge_w, relation_emb):
    e = edge_index.shape[1]
    gran = NW * CH * BATCH
    ep = ((e + gran - 1) // gran) * gran
    ni = inter_edge.shape[1]
    nip = ((ni + gran - 1) // gran) * gran

    head = edge_index[0]
    tail = edge_index[1]
    hseg = _pad_to(head, ep, N_ENT)
    hgat = _pad_to(head, ep, 0)
    tgat = _pad_to(tail, ep, 0)
    ept = ep // NW
    ridx = _pad_to((edge_type + NR - 2) % (NR - 1), ep, 0)
    ridx = ridx + (jnp.arange(ep, dtype=jnp.int32) // ept) * (NR - 1)
    relrep = jnp.tile(relation_emb, (NW, 1))

    dseg = _pad_to(inter_edge[0], nip, N_USR)
    sgat = _pad_to(inter_edge[1], nip, 0)
    w_i = _pad_to(inter_edge_w, nip, 0.0)

    ka = _make_ka(ep)
    kb = _make_kb(ep)
    kb2 = _make_kb2(ep)
    kc = _make_agg(ep, N_ENT)
    ke = _make_agg(nip, N_USR)

    entity = item_emb
    for _ in range(2):
        scores, segmax = ka(entity, hgat, hseg, tgat, ridx, relrep)
        ex, segsum = kb(scores, hseg, segmax)
        attn = kb2(ex, hseg, segsum)
        parts = kc(attn, hseg, tgat, entity)
        entity = _norm_combine(parts)

    uparts = ke(w_i, dseg, sgat, entity)
    user_out = _norm_combine(uparts)
    return (user_out, entity)


# speedup vs baseline: 2.8912x; 1.1171x over previous
import functools

import jax
import jax.numpy as jnp
from jax import lax
from jax.experimental import pallas as pl
from jax.experimental.pallas import tpu as pltpu
from jax.experimental.pallas import tpu_sc as plsc

NC, NS, L = 2, 16, 16
NW = NC * NS
N_ENT = 10000
N_USR = 10000
C = 128
NR = 12
NP = 10240
STRIPE = NP // NS
CH = 128
BATCH = 16

_mesh = plsc.VectorSubcoreMesh(core_axis_name="c", subcore_axis_name="s")
_params = pltpu.CompilerParams(needs_layout_passes=False)


def _pad_to(x, n, fill):
    return jnp.concatenate(
        [x, jnp.full((n - x.shape[0],), fill, x.dtype)]) if x.shape[0] != n else x


def _fill_loop(ref, n, value):
    def body(i, _):
        ref[pl.ds(i * L, L)] = jnp.full((L,), value, jnp.float32)
        return 0
    lax.fori_loop(0, n // L, body, 0, unroll=8)


def _scale_rows(rows, w_v):
    iota = lax.iota(jnp.int32, L)

    def grp(g, _):
        w16 = w_v[pl.ds(g * L, L)]
        for i in range(L):
            e = g * L + i
            bc = w16.at[jnp.full((L,), i, jnp.int32)].get(
                mode="promise_in_bounds")
            for j in range(C // L):
                sl = pl.ds(j * L, L)
                rows[e, sl] = rows[e, sl] * bc
        return 0
    lax.fori_loop(0, CH // L, grp, 0)


def _seg_run_max(k16, v16):
    ks, vs = plsc.sort_key_val(k16, v16)
    iota = lax.iota(jnp.int32, L)
    for d in (1, 2, 4, 8):
        idx = jnp.maximum(iota - d, 0)
        kp = ks.at[idx].get(mode="promise_in_bounds")
        vp = vs.at[idx].get(mode="promise_in_bounds")
        same = (kp == ks) & (iota >= d)
        vs = jnp.where(same, jnp.maximum(vs, vp), vs)
    nxt = jnp.minimum(iota + 1, L - 1)
    kn = ks.at[nxt].get(mode="promise_in_bounds")
    is_last = (kn != ks) | (iota == L - 1)
    return ks, vs, is_last


def _make_ka(ep):
    ept = ep // NW
    ncht = ept // CH

    @functools.partial(
        pl.kernel,
        out_type=(jax.ShapeDtypeStruct((ep,), jnp.float32),
                  jax.ShapeDtypeStruct((NC, NP), jnp.float32)),
        mesh=_mesh,
        compiler_params=_params,
        scratch_types=[
            pltpu.VMEM((ept,), jnp.int32),
            pltpu.VMEM((ept,), jnp.int32),
            pltpu.VMEM((ept,), jnp.int32),
            pltpu.VMEM((ept,), jnp.int32),
            pltpu.VMEM((CH, C), jnp.float32),
            pltpu.VMEM((CH, C), jnp.float32),
            pltpu.VMEM((CH, C), jnp.float32),
            pltpu.VMEM((CH,), jnp.float32),
            pltpu.VMEM((NP,), jnp.float32),
            pltpu.VMEM_SHARED((NS, NP), jnp.float32),
            pltpu.VMEM((STRIPE,), jnp.float32),
            pltpu.VMEM((STRIPE,), jnp.float32),
        ],
    )
    def ka(ent, hgat, hseg, tgat, ridx, relrep, scores_o, segmax_o,
           hgat_sl, hseg_sl, tgat_sl, ridx_sl, hrow, trow, rrow,
           score_v, maxarr, smax_sh, tmpbuf, resbuf):
        cid = lax.axis_index("c")
        sid = lax.axis_index("s")
        wid = cid * NS + sid
        base = wid * ept

        pltpu.sync_copy(hgat.at[pl.ds(base, ept)], hgat_sl)
        pltpu.sync_copy(hseg.at[pl.ds(base, ept)], hseg_sl)
        pltpu.sync_copy(tgat.at[pl.ds(base, ept)], tgat_sl)
        pltpu.sync_copy(ridx.at[pl.ds(base, ept)], ridx_sl)
        _fill_loop(maxarr, NP, -jnp.inf)
        iota = lax.iota(jnp.int32, L)

        def chunk_body(ci, _):
            pltpu.sync_copy(ent.at[hgat_sl.at[pl.ds(ci * CH, CH)]], hrow)
            pltpu.sync_copy(ent.at[tgat_sl.at[pl.ds(ci * CH, CH)]], trow)
            pltpu.sync_copy(relrep.at[ridx_sl.at[pl.ds(ci * CH, CH)]], rrow)

            def grp_scores(g, _):
                svec = jnp.zeros((L,), jnp.float32)
                for i in range(L):
                    e = g * L + i
                    acc = jnp.zeros((L,), jnp.float32)
                    for j in range(C // L):
                        slj = pl.ds(j * L, L)
                        acc = acc + hrow[e, slj] * trow[e, slj] * rrow[e, slj]
                    svec = jnp.where(iota == i, jnp.sum(acc), svec)
                score_v[pl.ds(g * L, L)] = jnp.exp(svec)
                return 0
            lax.fori_loop(0, CH // L, grp_scores, 0)
            pltpu.sync_copy(score_v, scores_o.at[pl.ds(base + ci * CH, CH)])

            def g_body(g, _):
                sl = pl.ds(g * L, L)
                k16 = hseg_sl[pl.ds(ci * CH + g * L, L)]
                ks, vs, is_last = _seg_run_max(k16, score_v[sl])
                cur = plsc.load_gather(maxarr, [ks], mask=is_last)
                plsc.store_scatter(maxarr, [ks],
                                   jnp.maximum(vs, cur), mask=is_last)
                return 0
            lax.fori_loop(0, CH // L, g_body, 0)
            return 0
        lax.fori_loop(0, ncht, chunk_body, 0)

        pltpu.sync_copy(maxarr, smax_sh.at[sid])
        plsc.subcore_barrier()
        _fill_loop(resbuf, STRIPE, -jnp.inf)
        for t in range(NS):
            pltpu.sync_copy(smax_sh.at[t, pl.ds(sid * STRIPE, STRIPE)], tmpbuf)

            def red_body(i, _):
                sl = pl.ds(i * L, L)
                resbuf[sl] = jnp.maximum(resbuf[sl], tmpbuf[sl])
                return 0
            lax.fori_loop(0, STRIPE // L, red_body, 0, unroll=8)
        pltpu.sync_copy(resbuf, segmax_o.at[cid, pl.ds(sid * STRIPE, STRIPE)])

    return ka


def _make_kb(ep):
    ept = ep // NW
    ncht = ept // CH

    @functools.partial(
        pl.kernel,
        out_type=(jax.ShapeDtypeStruct((ep,), jnp.float32),
                  jax.ShapeDtypeStruct((NC, NP), jnp.float32)),
        mesh=_mesh,
        compiler_params=_params,
        scratch_types=[
            pltpu.VMEM((ept,), jnp.float32),
            pltpu.VMEM((ept,), jnp.int32),
            pltpu.VMEM((ept,), jnp.float32),
            pltpu.VMEM((NP,), jnp.float32),
            pltpu.VMEM((NP,), jnp.float32),
            pltpu.VMEM((CH,), jnp.int32),
            pltpu.VMEM((STRIPE,), jnp.float32),
            pltpu.VMEM_SHARED((NP,), jnp.float32),
        ],
    )
    def kb(scores, hseg, segmax, ex_o, segsum_o,
           score_sl, hseg_sl, ex_sl, gmax, mx2, seg_cp, stripe_v, ssum_sh):
        cid = lax.axis_index("c")
        sid = lax.axis_index("s")
        wid = cid * NS + sid
        base = wid * ept
        pltpu.sync_copy(scores.at[pl.ds(base, ept)], score_sl)
        pltpu.sync_copy(hseg.at[pl.ds(base, ept)], hseg_sl)
        pltpu.sync_copy(segmax.at[0], gmax)
        pltpu.sync_copy(segmax.at[1], mx2)

        def mb(i, _):
            sl = pl.ds(i * L, L)
            gmax[sl] = jnp.maximum(gmax[sl], mx2[sl])
            return 0
        lax.fori_loop(0, NP // L, mb, 0, unroll=8)

        _fill_loop(stripe_v, STRIPE, 0.0)
        pltpu.sync_copy(stripe_v, ssum_sh.at[pl.ds(sid * STRIPE, STRIPE)])
        plsc.subcore_barrier()

        def chunk_body(ci, _):
            def g_body(g, _):
                o = ci * CH + g * L
                sl = pl.ds(g * L, L)
                k16 = hseg_sl[pl.ds(o, L)]
                mx = plsc.load_gather(gmax, [k16])
                ex_sl[pl.ds(o, L)] = jnp.exp(score_sl[pl.ds(o, L)] - mx)
                seg_cp[sl] = k16
                return 0
            lax.fori_loop(0, CH // L, g_body, 0, unroll=4)
            pltpu.sync_copy(ex_sl.at[pl.ds(ci * CH, CH)],
                            ssum_sh.at[seg_cp], add=True)
            return 0
        lax.fori_loop(0, ncht, chunk_body, 0)
        pltpu.sync_copy(ex_sl, ex_o.at[pl.ds(base, ept)])

        plsc.subcore_barrier()
        pltpu.sync_copy(ssum_sh.at[pl.ds(sid * STRIPE, STRIPE)],
                        segsum_o.at[cid, pl.ds(sid * STRIPE, STRIPE)])

    return kb


def _make_kb2(ep):
    ept = ep // NW
    ncht = ept // CH

    @functools.partial(
        pl.kernel,
        out_type=jax.ShapeDtypeStruct((ep,), jnp.float32),
        mesh=_mesh,
        compiler_params=_params,
        scratch_types=[
            pltpu.VMEM((ept,), jnp.float32),
            pltpu.VMEM((ept,), jnp.int32),
            pltpu.VMEM((NP,), jnp.float32),
            pltpu.VMEM((NP,), jnp.float32),
        ],
    )
    def kb2(ex_in, seg_in, segsum, attn_o, ex_sl, seg_sl, ssum, s2):
        cid = lax.axis_index("c")
        sid = lax.axis_index("s")
        wid = cid * NS + sid
        base = wid * ept
        pltpu.sync_copy(ex_in.at[pl.ds(base, ept)], ex_sl)
        pltpu.sync_copy(seg_in.at[pl.ds(base, ept)], seg_sl)
        pltpu.sync_copy(segsum.at[0], ssum)
        pltpu.sync_copy(segsum.at[1], s2)

        def mb(i, _):
            sl = pl.ds(i * L, L)
            ssum[sl] = ssum[sl] + s2[sl]
            return 0
        lax.fori_loop(0, NP // L, mb, 0, unroll=8)

        def g_body(g, _):
            sl = pl.ds(g * L, L)
            ss = plsc.load_gather(ssum, [seg_sl[sl]])
            ex_sl[sl] = ex_sl[sl] / ss
            return 0
        lax.fori_loop(0, ept // L, g_body, 0, unroll=8)
        pltpu.sync_copy(ex_sl, attn_o.at[pl.ds(base, ept)])

    return kb2


def _make_agg(ep, n_out):
    ept = ep // NW
    ncht = ept // CH
    nb = ncht // BATCH
    chb = BATCH * CH
    tail_rows = n_out - (NS - 1) * STRIPE

    @functools.partial(
        pl.kernel,
        out_type=jax.ShapeDtypeStruct((NC, n_out, C), jnp.float32),
        mesh=_mesh,
        compiler_params=_params,
        scratch_types=[
            pltpu.VMEM((chb,), jnp.float32),
            pltpu.VMEM((chb,), jnp.int32),
            pltpu.VMEM((chb,), jnp.int32),
            pltpu.VMEM((CH, C), jnp.float32),
            pltpu.VMEM((CH, C), jnp.float32),
            pltpu.VMEM((CH,), jnp.int32),
            pltpu.VMEM_SHARED((NP, C), jnp.float32),
            pltpu.SemaphoreType.DMA,
            pltpu.SemaphoreType.DMA,
            pltpu.SemaphoreType.DMA,
        ],
    )
    def kagg(w_in, seg_in, gat_in, ent, part_o,
             w_b, seg_b, gat_b, rb0, rb1, seg_cp, accum, g0, g1, zs):
        cid = lax.axis_index("c")
        sid = lax.axis_index("s")
        wid = cid * NS + sid
        base = wid * ept
        rbufs, gsems = (rb0, rb1), (g0, g1)

        def zrow(e, _):
            for j in range(C // L):
                rb0[e, pl.ds(j * L, L)] = jnp.zeros((L,), jnp.float32)
            return 0
        lax.fori_loop(0, CH, zrow, 0, unroll=4)
        for k in range(STRIPE // CH):
            pltpu.async_copy(rb0, accum.at[pl.ds(sid * STRIPE + k * CH, CH), :],
                             zs)
        for k in range(STRIPE // CH):
            pltpu.make_async_copy(
                rb0, accum.at[pl.ds(sid * STRIPE + k * CH, CH), :], zs).wait()
        plsc.subcore_barrier()

        def issue(lc, b):
            pltpu.async_copy(ent.at[gat_b.at[pl.ds(lc * CH, CH)]],
                             rbufs[b], gsems[b])

        def batch_body(bi, _):
            boff = base + bi * chb
            pltpu.sync_copy(w_in.at[pl.ds(boff, chb)], w_b)
            pltpu.sync_copy(seg_in.at[pl.ds(boff, chb)], seg_b)
            pltpu.sync_copy(gat_in.at[pl.ds(boff, chb)], gat_b)
            issue(0, 0)
            issue(1, 1)

            def pair_body(p, _):
                for b in range(2):
                    lc = p * 2 + b
                    pltpu.make_async_copy(
                        ent.at[gat_b.at[pl.ds(lc * CH, CH)]],
                        rbufs[b], gsems[b]).wait()

                    def g_body(g, _):
                        sl = pl.ds(g * L, L)
                        o = lc * CH + g * L
                        seg_cp[sl] = seg_b[pl.ds(o, L)]
                        return 0
                    lax.fori_loop(0, CH // L, g_body, 0, unroll=4)
                    _scale_rows(rbufs[b], w_b.at[pl.ds(lc * CH, CH)])
                    pltpu.sync_copy(rbufs[b], accum.at[seg_cp], add=True)

                    @pl.when(lc + 2 < BATCH)
                    def _():
                        issue(lc + 2, b)
                return 0
            lax.fori_loop(0, BATCH // 2, pair_body, 0)
            return 0
        lax.fori_loop(0, nb, batch_body, 0)

        plsc.subcore_barrier()

        @pl.when(sid < NS - 1)
        def _():
            pltpu.sync_copy(accum.at[pl.ds(sid * STRIPE, STRIPE), :],
                            part_o.at[cid, pl.ds(sid * STRIPE, STRIPE), :])

        @pl.when(sid == NS - 1)
        def _():
            pltpu.sync_copy(
                accum.at[pl.ds((NS - 1) * STRIPE, tail_rows), :],
                part_o.at[cid, pl.ds((NS - 1) * STRIPE, tail_rows), :])

    return kagg


def _norm_combine(parts):
    n = parts.shape[1]
    br = 400
    assert n % br == 0

    def body(p_ref, o_ref):
        s = p_ref[0] + p_ref[1]
        ss = jnp.sum(s * s, axis=1, keepdims=True)
        nrm = jnp.sqrt(ss)
        o_ref[...] = s / jnp.maximum(nrm, 1e-12)

    return pl.pallas_call(
        body,
        grid=(n // br,),
        in_specs=[pl.BlockSpec((NC, br, C), lambda i: (0, i, 0))],
        out_specs=pl.BlockSpec((br, C), lambda i: (i, 0)),
        out_shape=jax.ShapeDtypeStruct((n, C), jnp.float32),
    )(parts)


def kernel(user_emb, item_emb, edge_index, edge_type, inter_edge,
           inter_edge_w, relation_emb):
    e = edge_index.shape[1]
    gran = NW * CH * BATCH
    ep = ((e + gran - 1) // gran) * gran
    ni = inter_edge.shape[1]
    nip = ((ni + gran - 1) // gran) * gran

    head = edge_index[0]
    tail = edge_index[1]
    hseg = _pad_to(head, ep, N_ENT)
    hgat = _pad_to(head, ep, 0)
    tgat = _pad_to(tail, ep, 0)
    ept = ep // NW
    ridx = _pad_to((edge_type + NR - 2) % (NR - 1), ep, 0)
    ridx = ridx + (jnp.arange(ep, dtype=jnp.int32) // ept) * (NR - 1)
    relrep = jnp.tile(relation_emb, (NW, 1))

    dseg = _pad_to(inter_edge[0], nip, N_USR)
    sgat = _pad_to(inter_edge[1], nip, 0)
    w_i = _pad_to(inter_edge_w, nip, 0.0)

    ka = _make_ka(ep)
    kb = _make_kb(ep)
    kb2 = _make_kb2(ep)
    kc = _make_agg(ep, N_ENT)
    ke = _make_agg(nip, N_USR)

    entity = item_emb
    for _ in range(2):
        scores, segmax = ka(entity, hgat, hseg, tgat, ridx, relrep)
        ex, segsum = kb(scores, hseg, segmax)
        attn = kb2(ex, hseg, segsum)
        parts = kc(attn, hseg, tgat, entity)
        entity = _norm_combine(parts)

    uparts = ke(w_i, dseg, sgat, entity)
    user_out = _norm_combine(uparts)
    return (user_out, entity)
